# baseline (speedup 1.0000x reference)
import functools

import jax
import jax.numpy as jnp
from jax import lax
from jax.experimental import pallas as pl
from jax.experimental.pallas import tpu as pltpu
from jax.experimental.pallas import tpu_sc as plsc

_VOCAB = 1000
_HIDDEN = 128
_B = 4096
_SEQ = 20

_NC = 2
_NS = 16
_NW = _NC * _NS

_ROWS_W = _B // _NW


def _table_body(we_ref, w_ref, b_ref, out_ref):
    out_ref[...] = lax.dot_general(
        we_ref[...], w_ref[...],
        (((1,), (1,)), ((), ())),
        preferred_element_type=jnp.float32,
    ) + b_ref[...]


def _build_table(we, W, b):
    return pl.pallas_call(
        _table_body,
        out_shape=jax.ShapeDtypeStruct((_VOCAB, _VOCAB), jnp.float32),
    )(we, W, b.reshape(1, _VOCAB))


def _gather_body(table_hbm, idx_hbm, out_hbm, tab_s, idx_v, rows0, rows1,
                 gsem, ssem0, ssem1):
    cid = lax.axis_index("c")
    sid = lax.axis_index("s")
    wid = sid * _NC + cid
    base = wid * _ROWS_W
    rows = (rows0, rows1)
    ssem = (ssem0, ssem1)

    @pl.when(sid == 0)
    def _stage():
        pltpu.sync_copy(table_hbm, tab_s)

    pltpu.sync_copy(idx_hbm.at[wid], idx_v)
    plsc.subcore_barrier()

    def _do(j, p):
        pltpu.async_copy(tab_s.at[idx_v.at[j]], rows[p], gsem).wait()
        pltpu.async_copy(rows[p], out_hbm.at[base + j], ssem[p])

    def _drain(p):
        pltpu.make_async_copy(rows[p], out_hbm.at[0], ssem[p]).wait()

    _do(0, 0)
    _do(1, 1)

    @pl.loop(2, _ROWS_W, step=2)
    def _chunks(g):
        for p in range(2):
            _drain(p)
            _do(g + p, p)

    _drain(0)
    _drain(1)


@functools.partial(
    pl.kernel,
    out_type=jax.ShapeDtypeStruct((_B, _SEQ, _VOCAB), jnp.float32),
    mesh=plsc.VectorSubcoreMesh(core_axis_name="c", subcore_axis_name="s"),
    compiler_params=pltpu.CompilerParams(use_tc_tiling_on_sc=False),
    scratch_types=[
        pltpu.VMEM_SHARED((_VOCAB, _VOCAB), jnp.float32),
        pltpu.VMEM((_ROWS_W, _SEQ), jnp.int32),
        pltpu.VMEM((_SEQ, _VOCAB), jnp.float32),
        pltpu.VMEM((_SEQ, _VOCAB), jnp.float32),
        pltpu.SemaphoreType.DMA,
        pltpu.SemaphoreType.DMA,
        pltpu.SemaphoreType.DMA,
    ],
)
def _gather(table_hbm, idx_hbm, out_hbm, tab_s, idx_v, rows0, rows1,
            gsem, ssem0, ssem1):
    _gather_body(table_hbm, idx_hbm, out_hbm, tab_s, idx_v, rows0, rows1,
                 gsem, ssem0, ssem1)


def kernel(x, we, W, b):
    table = _build_table(we, W, b)
    idx = x.astype(jnp.int32).reshape(_NW, _ROWS_W, _SEQ)
    return _gather(table, idx)
